# Initial kernel scaffold; baseline (speedup 1.0000x reference)
import jax
import jax.numpy as jnp
from jax.experimental import pallas as pl


def kernel(idx, tok_emb, pos_emb):
    raise NotImplementedError("write your pallas kernel here")


# speedup vs baseline: 1.1432x; 1.1432x over previous
import functools

import jax
import jax.numpy as jnp
from jax import lax
from jax.experimental import pallas as pl
from jax.experimental.pallas import tpu as pltpu
from jax.experimental.pallas import tpu_sc as plsc

VOCAB = 100000
N_EMBD = 1024
B = 4
T = 2048

NC = 2
NS = 16
NW = NC * NS

N = B * T
PER_W = N // NW
C = 64
NCHUNK = PER_W // C


def _body(idx_hbm, tok_hbm, pos_hbm, out_hbm, idx_v, rows_v, sem):
    wid = lax.axis_index("s") * NC + lax.axis_index("c")
    base = wid * PER_W
    t0 = lax.rem(base, T)
    pltpu.sync_copy(idx_hbm.at[pl.ds(base, PER_W)], idx_v)
    for j in range(NCHUNK):
        pltpu.sync_copy(pos_hbm.at[pl.ds(t0 + j * C, C)], rows_v)
        pltpu.async_copy(
            tok_hbm.at[idx_v.at[pl.ds(j * C, C)]], rows_v, sem, add=True
        ).wait()
        pltpu.sync_copy(rows_v, out_hbm.at[pl.ds(base + j * C, C)])


_mesh = plsc.VectorSubcoreMesh(core_axis_name="c", subcore_axis_name="s")

_sc_call = functools.partial(
    pl.kernel,
    out_type=jax.ShapeDtypeStruct((N, N_EMBD), jnp.float32),
    mesh=_mesh,
    scratch_types=[
        pltpu.VMEM((PER_W,), jnp.int32),
        pltpu.VMEM((C, N_EMBD), jnp.float32),
        pltpu.SemaphoreType.DMA,
    ],
)(_body)


@jax.jit
def kernel(idx, tok_emb, pos_emb):
    idx_flat = idx.reshape(N).astype(jnp.int32)
    pos2d = pos_emb.reshape(T, N_EMBD)
    out = _sc_call(idx_flat, tok_emb, pos2d)
    return out.reshape(B, T, N_EMBD)


# speedup vs baseline: 1.2070x; 1.0559x over previous
import functools

import jax
import jax.numpy as jnp
from jax import lax
from jax.experimental import pallas as pl
from jax.experimental.pallas import tpu as pltpu
from jax.experimental.pallas import tpu_sc as plsc

VOCAB = 100000
N_EMBD = 1024
B = 4
T = 2048

NC = 2
NS = 16
NW = NC * NS

N = B * T
PER_W = N // NW
C = 32
NCHUNK = PER_W // C


def _body(idx_hbm, tok_hbm, pos_hbm, out_hbm, idx_v,
          rows0, rows1, sp0, sp1, sg0, sg1, ss0, ss1):
    wid = lax.axis_index("s") * NC + lax.axis_index("c")
    base = wid * PER_W
    t0 = lax.rem(base, T)
    pltpu.sync_copy(idx_hbm.at[pl.ds(base, PER_W)], idx_v)

    rows = [rows0, rows1]
    sem_p = [sp0, sp1]
    sem_g = [sg0, sg1]
    sem_s = [ss0, ss1]

    def issue_pos(j):
        return pltpu.async_copy(
            pos_hbm.at[pl.ds(t0 + j * C, C)], rows[j % 2], sem_p[j % 2])

    P = [None] * NCHUNK
    S = [None] * NCHUNK
    P[0] = issue_pos(0)
    for j in range(NCHUNK):
        b = j % 2
        P[j].wait()
        g = pltpu.async_copy(
            tok_hbm.at[idx_v.at[pl.ds(j * C, C)]], rows[b], sem_g[b],
            add=True)
        if j >= 1:
            S[j - 1].wait()
        if j + 1 < NCHUNK:
            P[j + 1] = issue_pos(j + 1)
        g.wait()
        S[j] = pltpu.async_copy(
            rows[b], out_hbm.at[pl.ds(base + j * C, C)], sem_s[b])
    S[NCHUNK - 1].wait()


_mesh = plsc.VectorSubcoreMesh(core_axis_name="c", subcore_axis_name="s")

_sc_call = functools.partial(
    pl.kernel,
    out_type=jax.ShapeDtypeStruct((N, N_EMBD), jnp.float32),
    mesh=_mesh,
    scratch_types=[
        pltpu.VMEM((PER_W,), jnp.int32),
        pltpu.VMEM((C, N_EMBD), jnp.float32),
        pltpu.VMEM((C, N_EMBD), jnp.float32),
    ] + [pltpu.SemaphoreType.DMA] * 6,
)(_body)


@jax.jit
def kernel(idx, tok_emb, pos_emb):
    idx_flat = idx.reshape(N).astype(jnp.int32)
    pos2d = pos_emb.reshape(T, N_EMBD)
    out = _sc_call(idx_flat, tok_emb, pos2d)
    return out.reshape(B, T, N_EMBD)


# speedup vs baseline: 1.2353x; 1.0234x over previous
import functools

import jax
import jax.numpy as jnp
from jax import lax
from jax.experimental import pallas as pl
from jax.experimental.pallas import tpu as pltpu
from jax.experimental.pallas import tpu_sc as plsc

VOCAB = 100000
N_EMBD = 1024
B = 4
T = 2048

NC = 2
NS = 16
NW = NC * NS

N = B * T
PER_W = N // NW
C = 16
NCHUNK = PER_W // C
K = 6
D = 3


def _body(idx_hbm, tok_hbm, pos_hbm, out_hbm, idx_v, *scratch):
    rows = list(scratch[:K])
    sem_p = list(scratch[K:2 * K])
    sem_g = list(scratch[2 * K:3 * K])
    sem_s = list(scratch[3 * K:4 * K])

    wid = lax.axis_index("s") * NC + lax.axis_index("c")
    base = wid * PER_W
    t0 = lax.rem(base, T)
    pltpu.sync_copy(idx_hbm.at[pl.ds(base, PER_W)], idx_v)

    def issue_pos(j):
        b = j % K
        return pltpu.async_copy(
            pos_hbm.at[pl.ds(t0 + j * C, C)], rows[b], sem_p[b])

    def issue_gather(j):
        b = j % K
        return pltpu.async_copy(
            tok_hbm.at[idx_v.at[pl.ds(j * C, C)]], rows[b], sem_g[b],
            add=True)

    def issue_store(j):
        b = j % K
        return pltpu.async_copy(
            rows[b], out_hbm.at[pl.ds(base + j * C, C)], sem_s[b])

    P = [None] * NCHUNK
    G = [None] * NCHUNK
    S = [None] * NCHUNK

    for i in range(D):
        P[i] = issue_pos(i)
    for j in range(NCHUNK):
        P[j].wait()
        G[j] = issue_gather(j)
        k = j + D
        if k < NCHUNK:
            if k - K >= 0:
                S[k - K].wait()
            P[k] = issue_pos(k)
        if j >= 1:
            G[j - 1].wait()
            S[j - 1] = issue_store(j - 1)
    G[NCHUNK - 1].wait()
    S[NCHUNK - 1] = issue_store(NCHUNK - 1)
    for j in range(NCHUNK - K, NCHUNK):
        S[j].wait()


_mesh = plsc.VectorSubcoreMesh(core_axis_name="c", subcore_axis_name="s")

_sc_call = functools.partial(
    pl.kernel,
    out_type=jax.ShapeDtypeStruct((N, N_EMBD), jnp.float32),
    mesh=_mesh,
    scratch_types=[pltpu.VMEM((PER_W,), jnp.int32)]
    + [pltpu.VMEM((C, N_EMBD), jnp.float32)] * K
    + [pltpu.SemaphoreType.DMA] * (3 * K),
)(_body)


@jax.jit
def kernel(idx, tok_emb, pos_emb):
    idx_flat = idx.reshape(N).astype(jnp.int32)
    pos2d = pos_emb.reshape(T, N_EMBD)
    out = _sc_call(idx_flat, tok_emb, pos2d)
    return out.reshape(B, T, N_EMBD)


# speedup vs baseline: 1.3360x; 1.0815x over previous
import functools

import jax
import jax.numpy as jnp
from jax import lax
from jax.experimental import pallas as pl
from jax.experimental.pallas import tpu as pltpu
from jax.experimental.pallas import tpu_sc as plsc

VOCAB = 100000
N_EMBD = 1024
B = 4
T = 2048

NC = 2
NS = 16
NW = NC * NS

N = B * T
TW = T // NW
PC = 32
C = 16
K = 4
NCHUNK = (B * TW) // C
GROUPS = C * N_EMBD // 16


def _body(idx_hbm, tok_hbm, pos_hbm, out_hbm, idx_v, pos_v, *scratch):
    rows = list(scratch[:K])
    sem_g = list(scratch[K:2 * K])
    sem_s = list(scratch[2 * K:3 * K])
    sem_pos = scratch[3 * K]

    wid = lax.axis_index("s") * NC + lax.axis_index("c")
    t0 = wid * TW
    for b in range(B):
        pltpu.sync_copy(idx_hbm.at[pl.ds(b * T + t0, TW)],
                        idx_v.at[pl.ds(b * TW, TW)])

    def offs(q):
        tc, j = divmod(q, 8)
        b, sub = divmod(j, 2)
        local = tc * PC + sub * C
        return b * TW + local, b * T + t0 + local, sub * C

    def issue_pos(tc):
        return pltpu.async_copy(
            pos_hbm.at[pl.ds(t0 + tc * PC, PC)], pos_v, sem_pos)

    def issue_gather(q):
        iofs, _, _ = offs(q)
        return pltpu.async_copy(
            tok_hbm.at[idx_v.at[pl.ds(iofs, C)]], rows[q % K], sem_g[q % K])

    def issue_store(q):
        _, oofs, _ = offs(q)
        return pltpu.async_copy(
            rows[q % K], out_hbm.at[pl.ds(oofs, C)], sem_s[q % K])

    def vpu_add(q):
        _, _, pofs = offs(q)
        buf = rows[q % K]

        @plsc.parallel_loop(0, GROUPS, unroll=8)
        def _(g):
            r = g >> 6
            c = (g & 63) * 16
            plsc.addupdate(buf.at[r, pl.ds(c, 16)],
                           pos_v[pofs + r, pl.ds(c, 16)])

    P = issue_pos(0)
    G = [None] * NCHUNK
    S = [None] * NCHUNK
    for i in range(K - 1):
        G[i] = issue_gather(i)
    P.wait()
    for q in range(NCHUNK):
        G[q].wait()
        if q == NCHUNK // 2:
            P.wait()
        vpu_add(q)
        S[q] = issue_store(q)
        if q == NCHUNK // 2 - 1:
            P = issue_pos(1)
        if q + K - 1 < NCHUNK:
            if q - 1 >= 0:
                S[q - 1].wait()
            G[q + K - 1] = issue_gather(q + K - 1)
    for q in range(NCHUNK - K, NCHUNK):
        S[q].wait()


_mesh = plsc.VectorSubcoreMesh(core_axis_name="c", subcore_axis_name="s")

_sc_call = functools.partial(
    pl.kernel,
    out_type=jax.ShapeDtypeStruct((N, N_EMBD), jnp.float32),
    mesh=_mesh,
    scratch_types=[
        pltpu.VMEM((B * TW,), jnp.int32),
        pltpu.VMEM((PC, N_EMBD), jnp.float32),
    ]
    + [pltpu.VMEM((C, N_EMBD), jnp.float32)] * K
    + [pltpu.SemaphoreType.DMA] * (2 * K + 1),
)(_body)


@jax.jit
def kernel(idx, tok_emb, pos_emb):
    idx_flat = idx.reshape(N).astype(jnp.int32)
    pos2d = pos_emb.reshape(T, N_EMBD)
    out = _sc_call(idx_flat, tok_emb, pos2d)
    return out.reshape(B, T, N_EMBD)
